# baseline (speedup 1.0000x reference)
import jax
import jax.numpy as jnp
from jax.experimental import pallas as pl

_HEADS = 16
_CODES = 512
_DIM = 64
_LB = 1024


def _vq_body(x_ref, c_ref, o_ref):
    xb = x_ref[0]
    cb = c_ref[0]
    xc = jax.lax.dot_general(
        xb, cb,
        dimension_numbers=(((1,), (1,)), ((), ())),
        preferred_element_type=jnp.float32,
    )
    x2 = jnp.sum(xb * xb, axis=1, keepdims=True)
    c2 = jnp.sum(cb * cb, axis=1)[None, :]
    dist = x2 - 2.0 * xc + c2
    minval = jnp.min(dist, axis=1, keepdims=True)
    iota = jax.lax.broadcasted_iota(jnp.int32, dist.shape, 1)
    first = jnp.min(jnp.where(dist == minval, iota, _CODES), axis=1, keepdims=True)
    o_ref[0] = (iota == first).astype(jnp.float32)


def kernel(x, c):
    b, h, l, d = x.shape
    s = c.shape[1]
    xr = x.reshape(b * h, l, d)
    grid = (b * h, l // _LB)
    out = pl.pallas_call(
        _vq_body,
        grid=grid,
        in_specs=[
            pl.BlockSpec((1, _LB, d), lambda i, j: (i, j, 0)),
            pl.BlockSpec((1, s, d), lambda i, j: (i % _HEADS, 0, 0)),
        ],
        out_specs=pl.BlockSpec((1, _LB, s), lambda i, j: (i, j, 0)),
        out_shape=jax.ShapeDtypeStruct((b * h, l, s), jnp.float32),
    )(xr, c)
    return (out.reshape(b, h, l, s), c)
